# baseline (speedup 1.0000x reference)
import functools

import jax
import jax.numpy as jnp
from jax import lax
from jax.experimental import pallas as pl
from jax.experimental.pallas import tpu as pltpu
from jax.experimental.pallas import tpu_sc as plsc

N = 10000
E = 320000
D = 128
H = 128
G = 128
C = 1

NC = 2
NS = 16
NW = NC * NS
CH = 128
CPB = 2
NB = 40
NCH = NB * CPB
EW = NCH * CH
N_PAD = 10240
RPT = N_PAD // NS
DUMP = N
BLK = 1024
NBLK = N_PAD // BLK

_mesh = plsc.VectorSubcoreMesh(core_axis_name="c", subcore_axis_name="s")


@functools.partial(
    pl.kernel,
    out_type=jax.ShapeDtypeStruct((NW, N_PAD), jnp.float32),
    mesh=_mesh,
    compiler_params=pltpu.CompilerParams(needs_layout_passes=False),
    scratch_types=[
        pltpu.VMEM((NB, CPB, CH), jnp.int32),
        pltpu.VMEM((N_PAD,), jnp.float32),
    ],
)
def _deg_sc(dst_hbm, deg_hbm, dst_v, deg_v):
    c = lax.axis_index("c")
    s = lax.axis_index("s")
    w = c * NS + s
    pltpu.sync_copy(dst_hbm.at[w], dst_v)
    zeros = jnp.zeros((16,), jnp.float32)

    @pl.loop(0, N_PAD // 16)
    def _(i):
        deg_v[pl.ds(i * 16, 16)] = zeros

    ones = jnp.ones((16,), jnp.float32)

    @pl.loop(0, NB)
    def _(j):
        for cc in range(CPB):
            for k in range(CH // 16):
                idx = dst_v[j, cc, pl.ds(k * 16, 16)]
                plsc.addupdate_scatter(deg_v, [idx], ones)

    pltpu.sync_copy(deg_v, deg_hbm.at[w])


@functools.partial(
    pl.kernel,
    out_type=jax.ShapeDtypeStruct((NC, N_PAD, H), jnp.float32),
    mesh=_mesh,
    scratch_types=[
        pltpu.VMEM((2, CPB, CH), jnp.int32),
        pltpu.VMEM((2, CPB, CH), jnp.int32),
        pltpu.VMEM((2, CH, H), jnp.float32),
        pltpu.VMEM_SHARED((N_PAD, H), jnp.float32),
        pltpu.SemaphoreType.DMA,
        pltpu.SemaphoreType.DMA,
        pltpu.SemaphoreType.DMA,
        pltpu.SemaphoreType.DMA,
    ],
)
def _seg_sc(u_hbm, src_hbm, dst_hbm, zer_hbm, out_hbm,
            src_v, dst_v, ring_v, acc_sh, g0, g1, i0, i1):
    c = lax.axis_index("c")
    s = lax.axis_index("s")
    w = c * NS + s
    gsem = (g0, g1)
    isem = (i0, i1)

    def copy_idx(blk, p):
        pltpu.async_copy(src_hbm.at[w, blk], src_v.at[p], isem[p])
        pltpu.async_copy(dst_hbm.at[w, blk], dst_v.at[p], isem[p])

    def wait_idx(p):
        pltpu.make_async_copy(src_hbm.at[0, 0], src_v.at[p], isem[p]).wait()
        pltpu.make_async_copy(dst_hbm.at[0, 0], dst_v.at[p], isem[p]).wait()

    def gather(p, cc, rb):
        pltpu.async_copy(u_hbm.at[src_v.at[p, cc]], ring_v.at[rb], gsem[rb])

    def wait_gather(rb):
        pltpu.make_async_copy(u_hbm.at[src_v.at[0, 0]], ring_v.at[rb],
                              gsem[rb]).wait()

    def scat(p, cc, rb):
        pltpu.sync_copy(ring_v.at[rb], acc_sh.at[dst_v.at[p, cc]], add=True)

    base = s * RPT
    for i in range(RPT // CH):
        pltpu.sync_copy(zer_hbm, acc_sh.at[pl.ds(base + i * CH, CH)])
    plsc.subcore_barrier()

    copy_idx(0, 0)
    wait_idx(0)
    gather(0, 0, 0)

    def block_body(blk, p):
        @pl.when(blk + 1 < NB)
        def _():
            copy_idx(blk + 1, 1 - p)
        for cc in range(CPB):
            if cc + 1 < CPB:
                gather(p, cc + 1, (cc + 1) % 2)
            wait_gather(cc % 2)
            scat(p, cc, cc % 2)

        @pl.when(blk + 1 < NB)
        def _():
            wait_idx(1 - p)
            gather(1 - p, 0, 0)

    @pl.loop(0, NB, step=2)
    def _(blk):
        block_body(blk, 0)
        block_body(blk + 1, 1)

    plsc.subcore_barrier()
    for i in range(RPT // CH):
        r = s * RPT + i * CH
        pltpu.sync_copy(acc_sh.at[pl.ds(r, CH)], out_hbm.at[c, pl.ds(r, CH)])


def _prep_body(deg_ref, x_ref, w0_ref, u_ref, dis_ref):
    deg = 1.0 + jnp.sum(deg_ref[...], axis=0)
    dis = lax.rsqrt(deg)[:, None]
    dis_ref[...] = dis
    u_ref[...] = dis * jnp.dot(x_ref[...], w0_ref[...],
                               preferred_element_type=jnp.float32)


_prep = pl.pallas_call(
    _prep_body,
    grid=(NBLK,),
    in_specs=[
        pl.BlockSpec((NW, BLK), lambda i: (0, i)),
        pl.BlockSpec((BLK, D), lambda i: (i, 0)),
        pl.BlockSpec((D, H), lambda i: (0, 0)),
    ],
    out_specs=[
        pl.BlockSpec((BLK, H), lambda i: (i, 0)),
        pl.BlockSpec((BLK, 1), lambda i: (i, 0)),
    ],
    out_shape=[
        jax.ShapeDtypeStruct((N_PAD, H), jnp.float32),
        jax.ShapeDtypeStruct((N_PAD, 1), jnp.float32),
    ],
)


def _merge_body(acc_ref, u_ref, dis_ref, b_ref, w_ref, un_ref):
    dis = dis_ref[...]
    t = acc_ref[0] + acc_ref[1] + u_ref[...]
    h = jnp.maximum(dis * t + b_ref[...], 0.0)
    un_ref[...] = dis * jnp.dot(h, w_ref[...],
                                preferred_element_type=jnp.float32)


_merge = pl.pallas_call(
    _merge_body,
    grid=(NBLK,),
    in_specs=[
        pl.BlockSpec((NC, BLK, H), lambda i: (0, i, 0)),
        pl.BlockSpec((BLK, H), lambda i: (i, 0)),
        pl.BlockSpec((BLK, 1), lambda i: (i, 0)),
        pl.BlockSpec((1, H), lambda i: (0, 0)),
        pl.BlockSpec((H, H), lambda i: (0, 0)),
    ],
    out_specs=pl.BlockSpec((BLK, H), lambda i: (i, 0)),
    out_shape=jax.ShapeDtypeStruct((N_PAD, H), jnp.float32),
)


def _final_body(acc_ref, u_ref, dis_ref, b_ref, batch_ref,
                wm0_ref, bm0_ref, wm1_ref, bm1_ref,
                out_ref, sums_ref, cnt_ref):
    i = pl.program_id(0)
    dis = dis_ref[...]
    h = jnp.maximum(dis * (acc_ref[0] + acc_ref[1] + u_ref[...]) + b_ref[...],
                    0.0)
    gids = lax.broadcasted_iota(jnp.int32, (BLK, G), 1)
    onehot = (batch_ref[...] == gids).astype(jnp.float32)
    ps = lax.dot_general(onehot, h, (((0,), (0,)), ((), ())),
                         preferred_element_type=jnp.float32,
                         precision=lax.Precision.HIGHEST)
    cs = lax.dot_general(onehot, jnp.ones((BLK, 1), jnp.float32),
                         (((0,), (0,)), ((), ())),
                         preferred_element_type=jnp.float32,
                         precision=lax.Precision.HIGHEST)

    @pl.when(i == 0)
    def _():
        sums_ref[...] = jnp.zeros_like(sums_ref)
        cnt_ref[...] = jnp.zeros_like(cnt_ref)

    sums_ref[...] += ps
    cnt_ref[...] += cs

    @pl.when(i == pl.num_programs(0) - 1)
    def _():
        pooled = sums_ref[...] / jnp.maximum(cnt_ref[...], 1.0)
        hm = jnp.maximum(
            jnp.dot(pooled, wm0_ref[...],
                    preferred_element_type=jnp.float32) + bm0_ref[...], 0.0)
        out_ref[...] = jnp.dot(hm, wm1_ref[...],
                               preferred_element_type=jnp.float32) + bm1_ref[...]


_final = pl.pallas_call(
    _final_body,
    grid=(NBLK,),
    in_specs=[
        pl.BlockSpec((NC, BLK, H), lambda i: (0, i, 0)),
        pl.BlockSpec((BLK, H), lambda i: (i, 0)),
        pl.BlockSpec((BLK, 1), lambda i: (i, 0)),
        pl.BlockSpec((1, H), lambda i: (0, 0)),
        pl.BlockSpec((BLK, 1), lambda i: (i, 0)),
        pl.BlockSpec((H, H), lambda i: (0, 0)),
        pl.BlockSpec((1, H), lambda i: (0, 0)),
        pl.BlockSpec((H, C), lambda i: (0, 0)),
        pl.BlockSpec((1, C), lambda i: (0, 0)),
    ],
    out_specs=pl.BlockSpec((G, C), lambda i: (0, 0)),
    out_shape=jax.ShapeDtypeStruct((G, C), jnp.float32),
    scratch_shapes=[
        pltpu.VMEM((G, H), jnp.float32),
        pltpu.VMEM((G, 1), jnp.float32),
    ],
)


def kernel(x, edge_index, batch, W0, b0, W1, b1, W2, b2, Wm0, bm0, Wm1, bm1):
    src = edge_index[0].astype(jnp.int32)
    dst = edge_index[1].astype(jnp.int32)
    pad_e = NW * EW - E
    src_p = jnp.concatenate(
        [src, jnp.zeros((pad_e,), jnp.int32)]).reshape(NW, NB, CPB, CH)
    dst_p = jnp.concatenate(
        [dst, jnp.full((pad_e,), DUMP, jnp.int32)]).reshape(NW, NB, CPB, CH)
    x_p = jnp.pad(x, ((0, N_PAD - N), (0, 0)))
    batch_p = jnp.concatenate(
        [batch.astype(jnp.int32),
         jnp.full((N_PAD - N,), G, jnp.int32)]).reshape(N_PAD, 1)
    zer = jnp.zeros((CH, H), jnp.float32)

    deg_parts = _deg_sc(dst_p)
    u0, dis = _prep(deg_parts, x_p, W0)
    acc = _seg_sc(u0, src_p, dst_p, zer)
    u1 = _merge(acc, u0, dis, b0.reshape(1, H), W1)
    acc = _seg_sc(u1, src_p, dst_p, zer)
    u2 = _merge(acc, u1, dis, b1.reshape(1, H), W2)
    acc = _seg_sc(u2, src_p, dst_p, zer)
    out = _final(acc, u2, dis, b2.reshape(1, H), batch_p,
                 Wm0, bm0.reshape(1, H), Wm1, bm1.reshape(1, C))
    return out


# speedup vs baseline: 2.9953x; 2.9953x over previous
import functools

import jax
import jax.numpy as jnp
from jax import lax
from jax.experimental import pallas as pl
from jax.experimental.pallas import tpu as pltpu
from jax.experimental.pallas import tpu_sc as plsc

N = 10000
E = 320000
D = 128
H = 128
G = 128
C = 1

NC = 2
NS = 16
NW = NC * NS
CH = 64
CPB = 4
NB = 40
NCH = NB * CPB
EW = NCH * CH
N_PAD = 10240
RPT = N_PAD // NS
DUMP = N
BLK = 1024
NBLK = N_PAD // BLK

_mesh = plsc.VectorSubcoreMesh(core_axis_name="c", subcore_axis_name="s")


@functools.partial(
    pl.kernel,
    out_type=jax.ShapeDtypeStruct((NW, N_PAD), jnp.float32),
    mesh=_mesh,
    compiler_params=pltpu.CompilerParams(needs_layout_passes=False),
    scratch_types=[
        pltpu.VMEM((NB, CPB, CH), jnp.int32),
        pltpu.VMEM((N_PAD,), jnp.float32),
    ],
)
def _deg_sc(dst_hbm, deg_hbm, dst_v, deg_v):
    c = lax.axis_index("c")
    s = lax.axis_index("s")
    w = c * NS + s
    pltpu.sync_copy(dst_hbm.at[w], dst_v)
    zeros = jnp.zeros((16,), jnp.float32)

    @pl.loop(0, N_PAD // 16)
    def _(i):
        deg_v[pl.ds(i * 16, 16)] = zeros

    ones = jnp.ones((16,), jnp.float32)

    @pl.loop(0, NB)
    def _(j):
        for cc in range(CPB):
            for k in range(CH // 16):
                idx = dst_v[j, cc, pl.ds(k * 16, 16)]
                plsc.addupdate_scatter(deg_v, [idx], ones)

    pltpu.sync_copy(deg_v, deg_hbm.at[w])


@functools.partial(
    pl.kernel,
    out_type=jax.ShapeDtypeStruct((NC, N_PAD, H), jnp.float32),
    mesh=_mesh,
    scratch_types=[
        pltpu.VMEM((2, CPB, CH), jnp.int32),
        pltpu.VMEM((2, CPB, CH), jnp.int32),
        pltpu.VMEM((2, CH, H), jnp.float32),
        pltpu.VMEM_SHARED((N_PAD, H), jnp.float32),
        pltpu.SemaphoreType.DMA,
        pltpu.SemaphoreType.DMA,
        pltpu.SemaphoreType.DMA,
        pltpu.SemaphoreType.DMA,
    ],
)
def _seg_sc(u_hbm, src_hbm, dst_hbm, zer_hbm, out_hbm,
            src_v, dst_v, ring_v, acc_sh, g0, g1, i0, i1):
    c = lax.axis_index("c")
    s = lax.axis_index("s")
    w = c * NS + s
    gsem = (g0, g1)
    isem = (i0, i1)

    def copy_idx(blk, p):
        pltpu.async_copy(src_hbm.at[w, blk], src_v.at[p], isem[p])
        pltpu.async_copy(dst_hbm.at[w, blk], dst_v.at[p], isem[p])

    def wait_idx(p):
        pltpu.make_async_copy(src_hbm.at[0, 0], src_v.at[p], isem[p]).wait()
        pltpu.make_async_copy(dst_hbm.at[0, 0], dst_v.at[p], isem[p]).wait()

    def gather(p, cc, rb):
        pltpu.async_copy(u_hbm.at[src_v.at[p, cc]], ring_v.at[rb], gsem[rb])

    def wait_gather(rb):
        pltpu.make_async_copy(u_hbm.at[src_v.at[0, 0]], ring_v.at[rb],
                              gsem[rb]).wait()

    def scat(p, cc, rb):
        pltpu.sync_copy(ring_v.at[rb], acc_sh.at[dst_v.at[p, cc]], add=True)

    base = s * RPT
    for i in range(RPT // CH):
        pltpu.sync_copy(zer_hbm, acc_sh.at[pl.ds(base + i * CH, CH)])
    plsc.subcore_barrier()

    copy_idx(0, 0)
    wait_idx(0)
    gather(0, 0, 0)

    def block_body(blk, p):
        @pl.when(blk + 1 < NB)
        def _():
            copy_idx(blk + 1, 1 - p)
        for cc in range(CPB):
            if cc + 1 < CPB:
                gather(p, cc + 1, (cc + 1) % 2)
            wait_gather(cc % 2)
            scat(p, cc, cc % 2)

        @pl.when(blk + 1 < NB)
        def _():
            wait_idx(1 - p)
            gather(1 - p, 0, 0)

    @pl.loop(0, NB, step=2)
    def _(blk):
        block_body(blk, 0)
        block_body(blk + 1, 1)

    plsc.subcore_barrier()
    for i in range(RPT // CH):
        r = s * RPT + i * CH
        pltpu.sync_copy(acc_sh.at[pl.ds(r, CH)], out_hbm.at[c, pl.ds(r, CH)])


def _prep_body(deg_ref, x_ref, w0_ref, u_ref, dis_ref):
    deg = 1.0 + jnp.sum(deg_ref[...], axis=0)
    dis = lax.rsqrt(deg)[:, None]
    dis_ref[...] = dis
    u_ref[...] = dis * jnp.dot(x_ref[...], w0_ref[...],
                               preferred_element_type=jnp.float32)


_prep = pl.pallas_call(
    _prep_body,
    grid=(NBLK,),
    in_specs=[
        pl.BlockSpec((NW, BLK), lambda i: (0, i)),
        pl.BlockSpec((BLK, D), lambda i: (i, 0)),
        pl.BlockSpec((D, H), lambda i: (0, 0)),
    ],
    out_specs=[
        pl.BlockSpec((BLK, H), lambda i: (i, 0)),
        pl.BlockSpec((BLK, 1), lambda i: (i, 0)),
    ],
    out_shape=[
        jax.ShapeDtypeStruct((N_PAD, H), jnp.float32),
        jax.ShapeDtypeStruct((N_PAD, 1), jnp.float32),
    ],
)


def _merge_body(acc_ref, u_ref, dis_ref, b_ref, w_ref, un_ref):
    dis = dis_ref[...]
    t = acc_ref[0] + acc_ref[1] + u_ref[...]
    h = jnp.maximum(dis * t + b_ref[...], 0.0)
    un_ref[...] = dis * jnp.dot(h, w_ref[...],
                                preferred_element_type=jnp.float32)


_merge = pl.pallas_call(
    _merge_body,
    grid=(NBLK,),
    in_specs=[
        pl.BlockSpec((NC, BLK, H), lambda i: (0, i, 0)),
        pl.BlockSpec((BLK, H), lambda i: (i, 0)),
        pl.BlockSpec((BLK, 1), lambda i: (i, 0)),
        pl.BlockSpec((1, H), lambda i: (0, 0)),
        pl.BlockSpec((H, H), lambda i: (0, 0)),
    ],
    out_specs=pl.BlockSpec((BLK, H), lambda i: (i, 0)),
    out_shape=jax.ShapeDtypeStruct((N_PAD, H), jnp.float32),
)


def _final_body(acc_ref, u_ref, dis_ref, b_ref, batch_ref,
                wm0_ref, bm0_ref, wm1_ref, bm1_ref,
                out_ref, sums_ref, cnt_ref):
    i = pl.program_id(0)
    dis = dis_ref[...]
    h = jnp.maximum(dis * (acc_ref[0] + acc_ref[1] + u_ref[...]) + b_ref[...],
                    0.0)
    gids = lax.broadcasted_iota(jnp.int32, (BLK, G), 1)
    onehot = (batch_ref[...] == gids).astype(jnp.float32)
    ps = lax.dot_general(onehot, h, (((0,), (0,)), ((), ())),
                         preferred_element_type=jnp.float32,
                         precision=lax.Precision.HIGHEST)
    cs = lax.dot_general(onehot, jnp.ones((BLK, 1), jnp.float32),
                         (((0,), (0,)), ((), ())),
                         preferred_element_type=jnp.float32,
                         precision=lax.Precision.HIGHEST)

    @pl.when(i == 0)
    def _():
        sums_ref[...] = jnp.zeros_like(sums_ref)
        cnt_ref[...] = jnp.zeros_like(cnt_ref)

    sums_ref[...] += ps
    cnt_ref[...] += cs

    @pl.when(i == pl.num_programs(0) - 1)
    def _():
        pooled = sums_ref[...] / jnp.maximum(cnt_ref[...], 1.0)
        hm = jnp.maximum(
            jnp.dot(pooled, wm0_ref[...],
                    preferred_element_type=jnp.float32) + bm0_ref[...], 0.0)
        out_ref[...] = jnp.dot(hm, wm1_ref[...],
                               preferred_element_type=jnp.float32) + bm1_ref[...]


_final = pl.pallas_call(
    _final_body,
    grid=(NBLK,),
    in_specs=[
        pl.BlockSpec((NC, BLK, H), lambda i: (0, i, 0)),
        pl.BlockSpec((BLK, H), lambda i: (i, 0)),
        pl.BlockSpec((BLK, 1), lambda i: (i, 0)),
        pl.BlockSpec((1, H), lambda i: (0, 0)),
        pl.BlockSpec((BLK, 1), lambda i: (i, 0)),
        pl.BlockSpec((H, H), lambda i: (0, 0)),
        pl.BlockSpec((1, H), lambda i: (0, 0)),
        pl.BlockSpec((H, C), lambda i: (0, 0)),
        pl.BlockSpec((1, C), lambda i: (0, 0)),
    ],
    out_specs=pl.BlockSpec((G, C), lambda i: (0, 0)),
    out_shape=jax.ShapeDtypeStruct((G, C), jnp.float32),
    scratch_shapes=[
        pltpu.VMEM((G, H), jnp.float32),
        pltpu.VMEM((G, 1), jnp.float32),
    ],
)


def kernel(x, edge_index, batch, W0, b0, W1, b1, W2, b2, Wm0, bm0, Wm1, bm1):
    src = edge_index[0].astype(jnp.int32)
    dst = edge_index[1].astype(jnp.int32)
    pad_e = NW * EW - E
    pad_i = jnp.arange(pad_e, dtype=jnp.int32)
    src_p = jnp.concatenate(
        [src, pad_i % N]).reshape(NW, NB, CPB, CH)
    dst_p = jnp.concatenate(
        [dst, DUMP + pad_i % (N_PAD - N)]).reshape(NW, NB, CPB, CH)
    x_p = jnp.pad(x, ((0, N_PAD - N), (0, 0)))
    batch_p = jnp.concatenate(
        [batch.astype(jnp.int32),
         jnp.full((N_PAD - N,), G, jnp.int32)]).reshape(N_PAD, 1)
    zer = jnp.zeros((CH, H), jnp.float32)

    deg_parts = _deg_sc(dst_p)
    u0, dis = _prep(deg_parts, x_p, W0)
    acc = _seg_sc(u0, src_p, dst_p, zer)
    u1 = _merge(acc, u0, dis, b0.reshape(1, H), W1)
    acc = _seg_sc(u1, src_p, dst_p, zer)
    u2 = _merge(acc, u1, dis, b1.reshape(1, H), W2)
    acc = _seg_sc(u2, src_p, dst_p, zer)
    out = _final(acc, u2, dis, b2.reshape(1, H), batch_p,
                 Wm0, bm0.reshape(1, H), Wm1, bm1.reshape(1, C))
    return out


# speedup vs baseline: 3.0835x; 1.0295x over previous
import functools

import jax
import jax.numpy as jnp
from jax import lax
from jax.experimental import pallas as pl
from jax.experimental.pallas import tpu as pltpu
from jax.experimental.pallas import tpu_sc as plsc

N = 10000
E = 320000
D = 128
H = 128
G = 128
C = 1

NC = 2
NS = 16
NW = NC * NS
CH = 128
CPB = 2
NB = 40
NCH = NB * CPB
EW = NCH * CH
N_PAD = 10240
RPT = N_PAD // NS
DUMP = N
BLK = 1024
NBLK = N_PAD // BLK

_mesh = plsc.VectorSubcoreMesh(core_axis_name="c", subcore_axis_name="s")


@functools.partial(
    pl.kernel,
    out_type=jax.ShapeDtypeStruct((NW, N_PAD), jnp.float32),
    mesh=_mesh,
    compiler_params=pltpu.CompilerParams(needs_layout_passes=False),
    scratch_types=[
        pltpu.VMEM((NB, CPB, CH), jnp.int32),
        pltpu.VMEM((N_PAD,), jnp.float32),
    ],
)
def _deg_sc(dst_hbm, deg_hbm, dst_v, deg_v):
    c = lax.axis_index("c")
    s = lax.axis_index("s")
    w = c * NS + s
    pltpu.sync_copy(dst_hbm.at[w], dst_v)
    zeros = jnp.zeros((16,), jnp.float32)

    @pl.loop(0, N_PAD // 16)
    def _(i):
        deg_v[pl.ds(i * 16, 16)] = zeros

    ones = jnp.ones((16,), jnp.float32)

    @pl.loop(0, NB)
    def _(j):
        for cc in range(CPB):
            for k in range(CH // 16):
                idx = dst_v[j, cc, pl.ds(k * 16, 16)]
                plsc.addupdate_scatter(deg_v, [idx], ones)

    pltpu.sync_copy(deg_v, deg_hbm.at[w])


@functools.partial(
    pl.kernel,
    out_type=jax.ShapeDtypeStruct((NC, N_PAD, H), jnp.float32),
    mesh=_mesh,
    scratch_types=[
        pltpu.VMEM((2, CPB, CH), jnp.int32),
        pltpu.VMEM((2, CPB, CH), jnp.int32),
        pltpu.VMEM((2, CH, H), jnp.float32),
        pltpu.VMEM_SHARED((N_PAD, H), jnp.float32),
        pltpu.SemaphoreType.DMA,
        pltpu.SemaphoreType.DMA,
        pltpu.SemaphoreType.DMA,
        pltpu.SemaphoreType.DMA,
    ],
)
def _seg_sc(u_hbm, src_hbm, dst_hbm, zer_hbm, out_hbm,
            src_v, dst_v, ring_v, acc_sh, g0, g1, i0, i1):
    c = lax.axis_index("c")
    s = lax.axis_index("s")
    w = c * NS + s
    gsem = (g0, g1)
    isem = (i0, i1)

    def copy_idx(blk, p):
        pltpu.async_copy(src_hbm.at[w, blk], src_v.at[p], isem[p])
        pltpu.async_copy(dst_hbm.at[w, blk], dst_v.at[p], isem[p])

    def wait_idx(p):
        pltpu.make_async_copy(src_hbm.at[0, 0], src_v.at[p], isem[p]).wait()
        pltpu.make_async_copy(dst_hbm.at[0, 0], dst_v.at[p], isem[p]).wait()

    def gather(p, cc, rb):
        pltpu.async_copy(u_hbm.at[src_v.at[p, cc]], ring_v.at[rb], gsem[rb])

    def wait_gather(rb):
        pltpu.make_async_copy(u_hbm.at[src_v.at[0, 0]], ring_v.at[rb],
                              gsem[rb]).wait()

    def scat(p, cc, rb):
        pltpu.sync_copy(ring_v.at[rb], acc_sh.at[dst_v.at[p, cc]], add=True)

    base = s * RPT
    for i in range(RPT // CH):
        pltpu.sync_copy(zer_hbm, acc_sh.at[pl.ds(base + i * CH, CH)])
    plsc.subcore_barrier()

    copy_idx(0, 0)
    wait_idx(0)
    gather(0, 0, 0)

    def block_body(blk, p):
        @pl.when(blk + 1 < NB)
        def _():
            copy_idx(blk + 1, 1 - p)
        for cc in range(CPB):
            if cc + 1 < CPB:
                gather(p, cc + 1, (cc + 1) % 2)
            wait_gather(cc % 2)
            scat(p, cc, cc % 2)

        @pl.when(blk + 1 < NB)
        def _():
            wait_idx(1 - p)
            gather(1 - p, 0, 0)

    @pl.loop(0, NB, step=2)
    def _(blk):
        block_body(blk, 0)
        block_body(blk + 1, 1)

    plsc.subcore_barrier()
    for i in range(RPT // CH):
        r = s * RPT + i * CH
        pltpu.sync_copy(acc_sh.at[pl.ds(r, CH)], out_hbm.at[c, pl.ds(r, CH)])


def _prep_body(deg_ref, x_ref, w0_ref, u_ref, dis_ref):
    deg = 1.0 + jnp.sum(deg_ref[...], axis=0)
    dis = lax.rsqrt(deg)[:, None]
    dis_ref[...] = dis
    u_ref[...] = dis * jnp.dot(x_ref[...], w0_ref[...],
                               preferred_element_type=jnp.float32)


_prep = pl.pallas_call(
    _prep_body,
    grid=(NBLK,),
    in_specs=[
        pl.BlockSpec((NW, BLK), lambda i: (0, i)),
        pl.BlockSpec((BLK, D), lambda i: (i, 0)),
        pl.BlockSpec((D, H), lambda i: (0, 0)),
    ],
    out_specs=[
        pl.BlockSpec((BLK, H), lambda i: (i, 0)),
        pl.BlockSpec((BLK, 1), lambda i: (i, 0)),
    ],
    out_shape=[
        jax.ShapeDtypeStruct((N_PAD, H), jnp.float32),
        jax.ShapeDtypeStruct((N_PAD, 1), jnp.float32),
    ],
)


def _merge_body(acc_ref, u_ref, dis_ref, b_ref, w_ref, un_ref):
    dis = dis_ref[...]
    t = acc_ref[0] + acc_ref[1] + u_ref[...]
    h = jnp.maximum(dis * t + b_ref[...], 0.0)
    un_ref[...] = dis * jnp.dot(h, w_ref[...],
                                preferred_element_type=jnp.float32)


_merge = pl.pallas_call(
    _merge_body,
    grid=(NBLK,),
    in_specs=[
        pl.BlockSpec((NC, BLK, H), lambda i: (0, i, 0)),
        pl.BlockSpec((BLK, H), lambda i: (i, 0)),
        pl.BlockSpec((BLK, 1), lambda i: (i, 0)),
        pl.BlockSpec((1, H), lambda i: (0, 0)),
        pl.BlockSpec((H, H), lambda i: (0, 0)),
    ],
    out_specs=pl.BlockSpec((BLK, H), lambda i: (i, 0)),
    out_shape=jax.ShapeDtypeStruct((N_PAD, H), jnp.float32),
)


def _final_body(acc_ref, u_ref, dis_ref, b_ref, batch_ref,
                wm0_ref, bm0_ref, wm1_ref, bm1_ref,
                out_ref, sums_ref, cnt_ref):
    i = pl.program_id(0)
    dis = dis_ref[...]
    h = jnp.maximum(dis * (acc_ref[0] + acc_ref[1] + u_ref[...]) + b_ref[...],
                    0.0)
    gids = lax.broadcasted_iota(jnp.int32, (BLK, G), 1)
    onehot = (batch_ref[...] == gids).astype(jnp.float32)
    ps = lax.dot_general(onehot, h, (((0,), (0,)), ((), ())),
                         preferred_element_type=jnp.float32,
                         precision=lax.Precision.HIGHEST)
    cs = lax.dot_general(onehot, jnp.ones((BLK, 1), jnp.float32),
                         (((0,), (0,)), ((), ())),
                         preferred_element_type=jnp.float32,
                         precision=lax.Precision.HIGHEST)

    @pl.when(i == 0)
    def _():
        sums_ref[...] = jnp.zeros_like(sums_ref)
        cnt_ref[...] = jnp.zeros_like(cnt_ref)

    sums_ref[...] += ps
    cnt_ref[...] += cs

    @pl.when(i == pl.num_programs(0) - 1)
    def _():
        pooled = sums_ref[...] / jnp.maximum(cnt_ref[...], 1.0)
        hm = jnp.maximum(
            jnp.dot(pooled, wm0_ref[...],
                    preferred_element_type=jnp.float32) + bm0_ref[...], 0.0)
        out_ref[...] = jnp.dot(hm, wm1_ref[...],
                               preferred_element_type=jnp.float32) + bm1_ref[...]


_final = pl.pallas_call(
    _final_body,
    grid=(NBLK,),
    in_specs=[
        pl.BlockSpec((NC, BLK, H), lambda i: (0, i, 0)),
        pl.BlockSpec((BLK, H), lambda i: (i, 0)),
        pl.BlockSpec((BLK, 1), lambda i: (i, 0)),
        pl.BlockSpec((1, H), lambda i: (0, 0)),
        pl.BlockSpec((BLK, 1), lambda i: (i, 0)),
        pl.BlockSpec((H, H), lambda i: (0, 0)),
        pl.BlockSpec((1, H), lambda i: (0, 0)),
        pl.BlockSpec((H, C), lambda i: (0, 0)),
        pl.BlockSpec((1, C), lambda i: (0, 0)),
    ],
    out_specs=pl.BlockSpec((G, C), lambda i: (0, 0)),
    out_shape=jax.ShapeDtypeStruct((G, C), jnp.float32),
    scratch_shapes=[
        pltpu.VMEM((G, H), jnp.float32),
        pltpu.VMEM((G, 1), jnp.float32),
    ],
)


def kernel(x, edge_index, batch, W0, b0, W1, b1, W2, b2, Wm0, bm0, Wm1, bm1):
    src = edge_index[0].astype(jnp.int32)
    dst = edge_index[1].astype(jnp.int32)
    pad_e = NW * EW - E
    pad_i = jnp.arange(pad_e, dtype=jnp.int32)
    src_p = jnp.concatenate(
        [src, pad_i % N]).reshape(NW, NB, CPB, CH)
    dst_p = jnp.concatenate(
        [dst, DUMP + pad_i % (N_PAD - N)]).reshape(NW, NB, CPB, CH)
    x_p = jnp.pad(x, ((0, N_PAD - N), (0, 0)))
    batch_p = jnp.concatenate(
        [batch.astype(jnp.int32),
         jnp.full((N_PAD - N,), G, jnp.int32)]).reshape(N_PAD, 1)
    zer = jnp.zeros((CH, H), jnp.float32)

    deg_parts = _deg_sc(dst_p)
    u0, dis = _prep(deg_parts, x_p, W0)
    acc = _seg_sc(u0, src_p, dst_p, zer)
    u1 = _merge(acc, u0, dis, b0.reshape(1, H), W1)
    acc = _seg_sc(u1, src_p, dst_p, zer)
    u2 = _merge(acc, u1, dis, b1.reshape(1, H), W2)
    acc = _seg_sc(u2, src_p, dst_p, zer)
    out = _final(acc, u2, dis, b2.reshape(1, H), batch_p,
                 Wm0, bm0.reshape(1, H), Wm1, bm1.reshape(1, C))
    return out


# speedup vs baseline: 3.4533x; 1.1199x over previous
import functools

import jax
import jax.numpy as jnp
from jax import lax
from jax.experimental import pallas as pl
from jax.experimental.pallas import tpu as pltpu
from jax.experimental.pallas import tpu_sc as plsc

N = 10000
E = 320000
D = 128
H = 128
G = 128
C = 1

NC = 2
NS = 16
NW = NC * NS
CH = 128
CPB = 4
NB = 20
NCH = NB * CPB
EW = NCH * CH
N_PAD = 10240
RPT = N_PAD // NS
DUMP = N
BLK = 1024
NBLK = N_PAD // BLK

_mesh = plsc.VectorSubcoreMesh(core_axis_name="c", subcore_axis_name="s")


@functools.partial(
    pl.kernel,
    out_type=jax.ShapeDtypeStruct((NW, N_PAD), jnp.float32),
    mesh=_mesh,
    compiler_params=pltpu.CompilerParams(needs_layout_passes=False),
    scratch_types=[
        pltpu.VMEM((NB, CPB, CH), jnp.int32),
        pltpu.VMEM((N_PAD,), jnp.float32),
    ],
)
def _deg_sc(dst_hbm, deg_hbm, dst_v, deg_v):
    c = lax.axis_index("c")
    s = lax.axis_index("s")
    w = c * NS + s
    pltpu.sync_copy(dst_hbm.at[w], dst_v)
    zeros = jnp.zeros((16,), jnp.float32)

    @pl.loop(0, N_PAD // 16)
    def _(i):
        deg_v[pl.ds(i * 16, 16)] = zeros

    ones = jnp.ones((16,), jnp.float32)

    @pl.loop(0, NB)
    def _(j):
        for cc in range(CPB):
            for k in range(CH // 16):
                idx = dst_v[j, cc, pl.ds(k * 16, 16)]
                plsc.addupdate_scatter(deg_v, [idx], ones)

    pltpu.sync_copy(deg_v, deg_hbm.at[w])


@functools.partial(
    pl.kernel,
    out_type=jax.ShapeDtypeStruct((NC, N_PAD, H), jnp.float32),
    mesh=_mesh,
    scratch_types=[
        pltpu.VMEM((2, CPB, CH), jnp.int32),
        pltpu.VMEM((2, CPB, CH), jnp.int32),
        pltpu.VMEM((2, CH, H), jnp.float32),
        pltpu.VMEM_SHARED((N_PAD, H), jnp.float32),
        pltpu.SemaphoreType.DMA,
        pltpu.SemaphoreType.DMA,
        pltpu.SemaphoreType.DMA,
        pltpu.SemaphoreType.DMA,
    ],
)
def _seg_sc(u_hbm, src_hbm, dst_hbm, zer_hbm, out_hbm,
            src_v, dst_v, ring_v, acc_sh, g0, g1, i0, i1):
    c = lax.axis_index("c")
    s = lax.axis_index("s")
    w = c * NS + s
    gsem = (g0, g1)
    isem = (i0, i1)

    def copy_idx(blk, p):
        pltpu.async_copy(src_hbm.at[w, blk], src_v.at[p], isem[p])
        pltpu.async_copy(dst_hbm.at[w, blk], dst_v.at[p], isem[p])

    def wait_idx(p):
        pltpu.make_async_copy(src_hbm.at[0, 0], src_v.at[p], isem[p]).wait()
        pltpu.make_async_copy(dst_hbm.at[0, 0], dst_v.at[p], isem[p]).wait()

    def gather(p, cc, rb):
        pltpu.async_copy(u_hbm.at[src_v.at[p, cc]], ring_v.at[rb], gsem[rb])

    def wait_gather(rb):
        pltpu.make_async_copy(u_hbm.at[src_v.at[0, 0]], ring_v.at[rb],
                              gsem[rb]).wait()

    def scat(p, cc, rb):
        pltpu.sync_copy(ring_v.at[rb], acc_sh.at[dst_v.at[p, cc]], add=True)

    base = s * RPT
    for i in range(RPT // CH):
        pltpu.sync_copy(zer_hbm, acc_sh.at[pl.ds(base + i * CH, CH)])
    plsc.subcore_barrier()

    copy_idx(0, 0)
    wait_idx(0)
    gather(0, 0, 0)

    def block_body(blk, p):
        @pl.when(blk + 1 < NB)
        def _():
            copy_idx(blk + 1, 1 - p)
        for cc in range(CPB):
            if cc + 1 < CPB:
                gather(p, cc + 1, (cc + 1) % 2)
            wait_gather(cc % 2)
            scat(p, cc, cc % 2)

        @pl.when(blk + 1 < NB)
        def _():
            wait_idx(1 - p)
            gather(1 - p, 0, 0)

    @pl.loop(0, NB, step=2)
    def _(blk):
        block_body(blk, 0)
        block_body(blk + 1, 1)

    plsc.subcore_barrier()
    for i in range(RPT // CH):
        r = s * RPT + i * CH
        pltpu.sync_copy(acc_sh.at[pl.ds(r, CH)], out_hbm.at[c, pl.ds(r, CH)])


def _prep_body(deg_ref, x_ref, w0_ref, u_ref, dis_ref):
    deg = 1.0 + jnp.sum(deg_ref[...], axis=0)
    dis = lax.rsqrt(deg)[:, None]
    dis_ref[...] = dis
    u_ref[...] = dis * jnp.dot(x_ref[...], w0_ref[...],
                               preferred_element_type=jnp.float32)


_prep = pl.pallas_call(
    _prep_body,
    grid=(NBLK,),
    in_specs=[
        pl.BlockSpec((NW, BLK), lambda i: (0, i)),
        pl.BlockSpec((BLK, D), lambda i: (i, 0)),
        pl.BlockSpec((D, H), lambda i: (0, 0)),
    ],
    out_specs=[
        pl.BlockSpec((BLK, H), lambda i: (i, 0)),
        pl.BlockSpec((BLK, 1), lambda i: (i, 0)),
    ],
    out_shape=[
        jax.ShapeDtypeStruct((N_PAD, H), jnp.float32),
        jax.ShapeDtypeStruct((N_PAD, 1), jnp.float32),
    ],
)


def _merge_body(acc_ref, u_ref, dis_ref, b_ref, w_ref, un_ref):
    dis = dis_ref[...]
    t = acc_ref[0] + acc_ref[1] + u_ref[...]
    h = jnp.maximum(dis * t + b_ref[...], 0.0)
    un_ref[...] = dis * jnp.dot(h, w_ref[...],
                                preferred_element_type=jnp.float32)


_merge = pl.pallas_call(
    _merge_body,
    grid=(NBLK,),
    in_specs=[
        pl.BlockSpec((NC, BLK, H), lambda i: (0, i, 0)),
        pl.BlockSpec((BLK, H), lambda i: (i, 0)),
        pl.BlockSpec((BLK, 1), lambda i: (i, 0)),
        pl.BlockSpec((1, H), lambda i: (0, 0)),
        pl.BlockSpec((H, H), lambda i: (0, 0)),
    ],
    out_specs=pl.BlockSpec((BLK, H), lambda i: (i, 0)),
    out_shape=jax.ShapeDtypeStruct((N_PAD, H), jnp.float32),
)


def _final_body(acc_ref, u_ref, dis_ref, b_ref, batch_ref,
                wm0_ref, bm0_ref, wm1_ref, bm1_ref,
                out_ref, sums_ref, cnt_ref):
    i = pl.program_id(0)
    dis = dis_ref[...]
    h = jnp.maximum(dis * (acc_ref[0] + acc_ref[1] + u_ref[...]) + b_ref[...],
                    0.0)
    gids = lax.broadcasted_iota(jnp.int32, (BLK, G), 1)
    onehot = (batch_ref[...] == gids).astype(jnp.float32)
    ps = lax.dot_general(onehot, h, (((0,), (0,)), ((), ())),
                         preferred_element_type=jnp.float32,
                         precision=lax.Precision.HIGHEST)
    cs = lax.dot_general(onehot, jnp.ones((BLK, 1), jnp.float32),
                         (((0,), (0,)), ((), ())),
                         preferred_element_type=jnp.float32,
                         precision=lax.Precision.HIGHEST)

    @pl.when(i == 0)
    def _():
        sums_ref[...] = jnp.zeros_like(sums_ref)
        cnt_ref[...] = jnp.zeros_like(cnt_ref)

    sums_ref[...] += ps
    cnt_ref[...] += cs

    @pl.when(i == pl.num_programs(0) - 1)
    def _():
        pooled = sums_ref[...] / jnp.maximum(cnt_ref[...], 1.0)
        hm = jnp.maximum(
            jnp.dot(pooled, wm0_ref[...],
                    preferred_element_type=jnp.float32) + bm0_ref[...], 0.0)
        out_ref[...] = jnp.dot(hm, wm1_ref[...],
                               preferred_element_type=jnp.float32) + bm1_ref[...]


_final = pl.pallas_call(
    _final_body,
    grid=(NBLK,),
    in_specs=[
        pl.BlockSpec((NC, BLK, H), lambda i: (0, i, 0)),
        pl.BlockSpec((BLK, H), lambda i: (i, 0)),
        pl.BlockSpec((BLK, 1), lambda i: (i, 0)),
        pl.BlockSpec((1, H), lambda i: (0, 0)),
        pl.BlockSpec((BLK, 1), lambda i: (i, 0)),
        pl.BlockSpec((H, H), lambda i: (0, 0)),
        pl.BlockSpec((1, H), lambda i: (0, 0)),
        pl.BlockSpec((H, C), lambda i: (0, 0)),
        pl.BlockSpec((1, C), lambda i: (0, 0)),
    ],
    out_specs=pl.BlockSpec((G, C), lambda i: (0, 0)),
    out_shape=jax.ShapeDtypeStruct((G, C), jnp.float32),
    scratch_shapes=[
        pltpu.VMEM((G, H), jnp.float32),
        pltpu.VMEM((G, 1), jnp.float32),
    ],
)


def kernel(x, edge_index, batch, W0, b0, W1, b1, W2, b2, Wm0, bm0, Wm1, bm1):
    src = edge_index[0].astype(jnp.int32)
    dst = edge_index[1].astype(jnp.int32)
    pad_e = NW * EW - E
    pad_i = jnp.arange(pad_e, dtype=jnp.int32)
    src_p = jnp.concatenate(
        [src, pad_i % N]).reshape(NW, NB, CPB, CH)
    dst_p = jnp.concatenate(
        [dst, DUMP + pad_i % (N_PAD - N)]).reshape(NW, NB, CPB, CH)
    x_p = jnp.pad(x, ((0, N_PAD - N), (0, 0)))
    batch_p = jnp.concatenate(
        [batch.astype(jnp.int32),
         jnp.full((N_PAD - N,), G, jnp.int32)]).reshape(N_PAD, 1)
    zer = jnp.zeros((CH, H), jnp.float32)

    deg_parts = _deg_sc(dst_p)
    u0, dis = _prep(deg_parts, x_p, W0)
    acc = _seg_sc(u0, src_p, dst_p, zer)
    u1 = _merge(acc, u0, dis, b0.reshape(1, H), W1)
    acc = _seg_sc(u1, src_p, dst_p, zer)
    u2 = _merge(acc, u1, dis, b1.reshape(1, H), W2)
    acc = _seg_sc(u2, src_p, dst_p, zer)
    out = _final(acc, u2, dis, b2.reshape(1, H), batch_p,
                 Wm0, bm0.reshape(1, H), Wm1, bm1.reshape(1, C))
    return out


# speedup vs baseline: 3.6709x; 1.0630x over previous
import functools

import jax
import jax.numpy as jnp
from jax import lax
from jax.experimental import pallas as pl
from jax.experimental.pallas import tpu as pltpu
from jax.experimental.pallas import tpu_sc as plsc

N = 10000
E = 320000
D = 128
H = 128
G = 128
C = 1

NC = 2
NS = 16
NW = NC * NS
CH = 128
CPB = 8
NB = 10
NCH = NB * CPB
EW = NCH * CH
N_PAD = 10240
RPT = N_PAD // NS
DUMP = N
BLK = 1024
NBLK = N_PAD // BLK

_mesh = plsc.VectorSubcoreMesh(core_axis_name="c", subcore_axis_name="s")


@functools.partial(
    pl.kernel,
    out_type=jax.ShapeDtypeStruct((NW, N_PAD), jnp.float32),
    mesh=_mesh,
    compiler_params=pltpu.CompilerParams(needs_layout_passes=False),
    scratch_types=[
        pltpu.VMEM((NB, CPB, CH), jnp.int32),
        pltpu.VMEM((N_PAD,), jnp.float32),
    ],
)
def _deg_sc(dst_hbm, deg_hbm, dst_v, deg_v):
    c = lax.axis_index("c")
    s = lax.axis_index("s")
    w = c * NS + s
    pltpu.sync_copy(dst_hbm.at[w], dst_v)
    zeros = jnp.zeros((16,), jnp.float32)

    @pl.loop(0, N_PAD // 16)
    def _(i):
        deg_v[pl.ds(i * 16, 16)] = zeros

    ones = jnp.ones((16,), jnp.float32)

    @pl.loop(0, NB)
    def _(j):
        for cc in range(CPB):
            for k in range(CH // 16):
                idx = dst_v[j, cc, pl.ds(k * 16, 16)]
                plsc.addupdate_scatter(deg_v, [idx], ones)

    pltpu.sync_copy(deg_v, deg_hbm.at[w])


@functools.partial(
    pl.kernel,
    out_type=jax.ShapeDtypeStruct((NC, N_PAD, H), jnp.float32),
    mesh=_mesh,
    scratch_types=[
        pltpu.VMEM((2, CPB, CH), jnp.int32),
        pltpu.VMEM((2, CPB, CH), jnp.int32),
        pltpu.VMEM((2, CH, H), jnp.float32),
        pltpu.VMEM_SHARED((N_PAD, H), jnp.float32),
        pltpu.SemaphoreType.DMA,
        pltpu.SemaphoreType.DMA,
        pltpu.SemaphoreType.DMA,
        pltpu.SemaphoreType.DMA,
    ],
)
def _seg_sc(u_hbm, src_hbm, dst_hbm, zer_hbm, out_hbm,
            src_v, dst_v, ring_v, acc_sh, g0, g1, i0, i1):
    c = lax.axis_index("c")
    s = lax.axis_index("s")
    w = c * NS + s
    gsem = (g0, g1)
    isem = (i0, i1)

    def copy_idx(blk, p):
        pltpu.async_copy(src_hbm.at[w, blk], src_v.at[p], isem[p])
        pltpu.async_copy(dst_hbm.at[w, blk], dst_v.at[p], isem[p])

    def wait_idx(p):
        pltpu.make_async_copy(src_hbm.at[0, 0], src_v.at[p], isem[p]).wait()
        pltpu.make_async_copy(dst_hbm.at[0, 0], dst_v.at[p], isem[p]).wait()

    def gather(p, cc, rb):
        pltpu.async_copy(u_hbm.at[src_v.at[p, cc]], ring_v.at[rb], gsem[rb])

    def wait_gather(rb):
        pltpu.make_async_copy(u_hbm.at[src_v.at[0, 0]], ring_v.at[rb],
                              gsem[rb]).wait()

    def scat(p, cc, rb):
        pltpu.sync_copy(ring_v.at[rb], acc_sh.at[dst_v.at[p, cc]], add=True)

    base = s * RPT
    for i in range(RPT // CH):
        pltpu.sync_copy(zer_hbm, acc_sh.at[pl.ds(base + i * CH, CH)])
    plsc.subcore_barrier()

    copy_idx(0, 0)
    wait_idx(0)
    gather(0, 0, 0)

    def block_body(blk, p):
        @pl.when(blk + 1 < NB)
        def _():
            copy_idx(blk + 1, 1 - p)
        for cc in range(CPB):
            if cc + 1 < CPB:
                gather(p, cc + 1, (cc + 1) % 2)
            wait_gather(cc % 2)
            scat(p, cc, cc % 2)

        @pl.when(blk + 1 < NB)
        def _():
            wait_idx(1 - p)
            gather(1 - p, 0, 0)

    @pl.loop(0, NB, step=2)
    def _(blk):
        block_body(blk, 0)
        block_body(blk + 1, 1)

    plsc.subcore_barrier()
    for i in range(RPT // CH):
        r = s * RPT + i * CH
        pltpu.sync_copy(acc_sh.at[pl.ds(r, CH)], out_hbm.at[c, pl.ds(r, CH)])


def _prep_body(deg_ref, x_ref, w0_ref, u_ref, dis_ref):
    deg = 1.0 + jnp.sum(deg_ref[...], axis=0)
    dis = lax.rsqrt(deg)[:, None]
    dis_ref[...] = dis
    u_ref[...] = dis * jnp.dot(x_ref[...], w0_ref[...],
                               preferred_element_type=jnp.float32)


_prep = pl.pallas_call(
    _prep_body,
    grid=(NBLK,),
    in_specs=[
        pl.BlockSpec((NW, BLK), lambda i: (0, i)),
        pl.BlockSpec((BLK, D), lambda i: (i, 0)),
        pl.BlockSpec((D, H), lambda i: (0, 0)),
    ],
    out_specs=[
        pl.BlockSpec((BLK, H), lambda i: (i, 0)),
        pl.BlockSpec((BLK, 1), lambda i: (i, 0)),
    ],
    out_shape=[
        jax.ShapeDtypeStruct((N_PAD, H), jnp.float32),
        jax.ShapeDtypeStruct((N_PAD, 1), jnp.float32),
    ],
)


def _merge_body(acc_ref, u_ref, dis_ref, b_ref, w_ref, un_ref):
    dis = dis_ref[...]
    t = acc_ref[0] + acc_ref[1] + u_ref[...]
    h = jnp.maximum(dis * t + b_ref[...], 0.0)
    un_ref[...] = dis * jnp.dot(h, w_ref[...],
                                preferred_element_type=jnp.float32)


_merge = pl.pallas_call(
    _merge_body,
    grid=(NBLK,),
    in_specs=[
        pl.BlockSpec((NC, BLK, H), lambda i: (0, i, 0)),
        pl.BlockSpec((BLK, H), lambda i: (i, 0)),
        pl.BlockSpec((BLK, 1), lambda i: (i, 0)),
        pl.BlockSpec((1, H), lambda i: (0, 0)),
        pl.BlockSpec((H, H), lambda i: (0, 0)),
    ],
    out_specs=pl.BlockSpec((BLK, H), lambda i: (i, 0)),
    out_shape=jax.ShapeDtypeStruct((N_PAD, H), jnp.float32),
)


def _final_body(acc_ref, u_ref, dis_ref, b_ref, batch_ref,
                wm0_ref, bm0_ref, wm1_ref, bm1_ref,
                out_ref, sums_ref, cnt_ref):
    i = pl.program_id(0)
    dis = dis_ref[...]
    h = jnp.maximum(dis * (acc_ref[0] + acc_ref[1] + u_ref[...]) + b_ref[...],
                    0.0)
    gids = lax.broadcasted_iota(jnp.int32, (BLK, G), 1)
    onehot = (batch_ref[...] == gids).astype(jnp.float32)
    ps = lax.dot_general(onehot, h, (((0,), (0,)), ((), ())),
                         preferred_element_type=jnp.float32,
                         precision=lax.Precision.HIGHEST)
    cs = lax.dot_general(onehot, jnp.ones((BLK, 1), jnp.float32),
                         (((0,), (0,)), ((), ())),
                         preferred_element_type=jnp.float32,
                         precision=lax.Precision.HIGHEST)

    @pl.when(i == 0)
    def _():
        sums_ref[...] = jnp.zeros_like(sums_ref)
        cnt_ref[...] = jnp.zeros_like(cnt_ref)

    sums_ref[...] += ps
    cnt_ref[...] += cs

    @pl.when(i == pl.num_programs(0) - 1)
    def _():
        pooled = sums_ref[...] / jnp.maximum(cnt_ref[...], 1.0)
        hm = jnp.maximum(
            jnp.dot(pooled, wm0_ref[...],
                    preferred_element_type=jnp.float32) + bm0_ref[...], 0.0)
        out_ref[...] = jnp.dot(hm, wm1_ref[...],
                               preferred_element_type=jnp.float32) + bm1_ref[...]


_final = pl.pallas_call(
    _final_body,
    grid=(NBLK,),
    in_specs=[
        pl.BlockSpec((NC, BLK, H), lambda i: (0, i, 0)),
        pl.BlockSpec((BLK, H), lambda i: (i, 0)),
        pl.BlockSpec((BLK, 1), lambda i: (i, 0)),
        pl.BlockSpec((1, H), lambda i: (0, 0)),
        pl.BlockSpec((BLK, 1), lambda i: (i, 0)),
        pl.BlockSpec((H, H), lambda i: (0, 0)),
        pl.BlockSpec((1, H), lambda i: (0, 0)),
        pl.BlockSpec((H, C), lambda i: (0, 0)),
        pl.BlockSpec((1, C), lambda i: (0, 0)),
    ],
    out_specs=pl.BlockSpec((G, C), lambda i: (0, 0)),
    out_shape=jax.ShapeDtypeStruct((G, C), jnp.float32),
    scratch_shapes=[
        pltpu.VMEM((G, H), jnp.float32),
        pltpu.VMEM((G, 1), jnp.float32),
    ],
)


def kernel(x, edge_index, batch, W0, b0, W1, b1, W2, b2, Wm0, bm0, Wm1, bm1):
    src = edge_index[0].astype(jnp.int32)
    dst = edge_index[1].astype(jnp.int32)
    pad_e = NW * EW - E
    pad_i = jnp.arange(pad_e, dtype=jnp.int32)
    src_p = jnp.concatenate(
        [src, pad_i % N]).reshape(NW, NB, CPB, CH)
    dst_p = jnp.concatenate(
        [dst, DUMP + pad_i % (N_PAD - N)]).reshape(NW, NB, CPB, CH)
    x_p = jnp.pad(x, ((0, N_PAD - N), (0, 0)))
    batch_p = jnp.concatenate(
        [batch.astype(jnp.int32),
         jnp.full((N_PAD - N,), G, jnp.int32)]).reshape(N_PAD, 1)
    zer = jnp.zeros((CH, H), jnp.float32)

    deg_parts = _deg_sc(dst_p)
    u0, dis = _prep(deg_parts, x_p, W0)
    acc = _seg_sc(u0, src_p, dst_p, zer)
    u1 = _merge(acc, u0, dis, b0.reshape(1, H), W1)
    acc = _seg_sc(u1, src_p, dst_p, zer)
    u2 = _merge(acc, u1, dis, b1.reshape(1, H), W2)
    acc = _seg_sc(u2, src_p, dst_p, zer)
    out = _final(acc, u2, dis, b2.reshape(1, H), batch_p,
                 Wm0, bm0.reshape(1, H), Wm1, bm1.reshape(1, C))
    return out


# speedup vs baseline: 3.7462x; 1.0205x over previous
import functools

import jax
import jax.numpy as jnp
from jax import lax
from jax.experimental import pallas as pl
from jax.experimental.pallas import tpu as pltpu
from jax.experimental.pallas import tpu_sc as plsc

N = 10000
E = 320000
D = 128
H = 128
G = 128
C = 1

NC = 2
NS = 16
NW = NC * NS
CH = 128
CPB = 20
NB = 4
NCH = NB * CPB
EW = NCH * CH
N_PAD = 10240
RPT = N_PAD // NS
DUMP = N
BLK = 1024
NBLK = N_PAD // BLK

_mesh = plsc.VectorSubcoreMesh(core_axis_name="c", subcore_axis_name="s")


@functools.partial(
    pl.kernel,
    out_type=jax.ShapeDtypeStruct((NW, N_PAD), jnp.float32),
    mesh=_mesh,
    compiler_params=pltpu.CompilerParams(needs_layout_passes=False),
    scratch_types=[
        pltpu.VMEM((EW,), jnp.int32),
        pltpu.VMEM((N_PAD,), jnp.float32),
    ],
)
def _deg_sc(dst_hbm, deg_hbm, dst_v, deg_v):
    c = lax.axis_index("c")
    s = lax.axis_index("s")
    w = c * NS + s
    pltpu.sync_copy(dst_hbm.at[w], dst_v)
    zeros = jnp.zeros((16,), jnp.float32)

    @pl.loop(0, N_PAD // 16)
    def _(i):
        deg_v[pl.ds(i * 16, 16)] = zeros

    ones = jnp.ones((16,), jnp.float32)

    @pl.loop(0, EW // 16)
    def _(i):
        idx = dst_v[pl.ds(i * 16, 16)]
        plsc.addupdate_scatter(deg_v, [idx], ones)

    pltpu.sync_copy(deg_v, deg_hbm.at[w])


@functools.partial(
    pl.kernel,
    out_type=jax.ShapeDtypeStruct((NC, N_PAD, H), jnp.float32),
    mesh=_mesh,
    scratch_types=[
        pltpu.VMEM((2, CPB, CH), jnp.int32),
        pltpu.VMEM((2, CPB, CH), jnp.int32),
        pltpu.VMEM((2, CH, H), jnp.float32),
        pltpu.VMEM_SHARED((N_PAD, H), jnp.float32),
        pltpu.SemaphoreType.DMA,
        pltpu.SemaphoreType.DMA,
        pltpu.SemaphoreType.DMA,
        pltpu.SemaphoreType.DMA,
    ],
)
def _seg_sc(u_hbm, src_hbm, dst_hbm, zer_hbm, out_hbm,
            src_v, dst_v, ring_v, acc_sh, g0, g1, i0, i1):
    c = lax.axis_index("c")
    s = lax.axis_index("s")
    w = c * NS + s
    gsem = (g0, g1)
    isem = (i0, i1)

    def copy_idx(blk, p):
        pltpu.async_copy(src_hbm.at[w, blk], src_v.at[p], isem[p])
        pltpu.async_copy(dst_hbm.at[w, blk], dst_v.at[p], isem[p])

    def wait_idx(p):
        pltpu.make_async_copy(src_hbm.at[0, 0], src_v.at[p], isem[p]).wait()
        pltpu.make_async_copy(dst_hbm.at[0, 0], dst_v.at[p], isem[p]).wait()

    def gather(p, cc, rb):
        pltpu.async_copy(u_hbm.at[src_v.at[p, cc]], ring_v.at[rb], gsem[rb])

    def wait_gather(rb):
        pltpu.make_async_copy(u_hbm.at[src_v.at[0, 0]], ring_v.at[rb],
                              gsem[rb]).wait()

    def scat(p, cc, rb):
        pltpu.sync_copy(ring_v.at[rb], acc_sh.at[dst_v.at[p, cc]], add=True)

    base = s * RPT
    for i in range(RPT // CH):
        pltpu.sync_copy(zer_hbm, acc_sh.at[pl.ds(base + i * CH, CH)])
    plsc.subcore_barrier()

    copy_idx(0, 0)
    wait_idx(0)
    gather(0, 0, 0)

    def block_body(blk, p):
        @pl.when(blk + 1 < NB)
        def _():
            copy_idx(blk + 1, 1 - p)
        for cc in range(CPB):
            if cc + 1 < CPB:
                gather(p, cc + 1, (cc + 1) % 2)
            wait_gather(cc % 2)
            scat(p, cc, cc % 2)

        @pl.when(blk + 1 < NB)
        def _():
            wait_idx(1 - p)
            gather(1 - p, 0, 0)

    @pl.loop(0, NB, step=2)
    def _(blk):
        block_body(blk, 0)
        block_body(blk + 1, 1)

    plsc.subcore_barrier()
    for i in range(RPT // CH):
        r = s * RPT + i * CH
        pltpu.sync_copy(acc_sh.at[pl.ds(r, CH)], out_hbm.at[c, pl.ds(r, CH)])


def _prep_body(deg_ref, x_ref, w0_ref, u_ref, dis_ref):
    deg = 1.0 + jnp.sum(deg_ref[...], axis=0)
    dis = lax.rsqrt(deg)[:, None]
    dis_ref[...] = dis
    u_ref[...] = dis * jnp.dot(x_ref[...], w0_ref[...],
                               preferred_element_type=jnp.float32)


_prep = pl.pallas_call(
    _prep_body,
    grid=(NBLK,),
    in_specs=[
        pl.BlockSpec((NW, BLK), lambda i: (0, i)),
        pl.BlockSpec((BLK, D), lambda i: (i, 0)),
        pl.BlockSpec((D, H), lambda i: (0, 0)),
    ],
    out_specs=[
        pl.BlockSpec((BLK, H), lambda i: (i, 0)),
        pl.BlockSpec((BLK, 1), lambda i: (i, 0)),
    ],
    out_shape=[
        jax.ShapeDtypeStruct((N_PAD, H), jnp.float32),
        jax.ShapeDtypeStruct((N_PAD, 1), jnp.float32),
    ],
)


def _merge_body(acc_ref, u_ref, dis_ref, b_ref, w_ref, un_ref):
    dis = dis_ref[...]
    t = acc_ref[0] + acc_ref[1] + u_ref[...]
    h = jnp.maximum(dis * t + b_ref[...], 0.0)
    un_ref[...] = dis * jnp.dot(h, w_ref[...],
                                preferred_element_type=jnp.float32)


_merge = pl.pallas_call(
    _merge_body,
    grid=(NBLK,),
    in_specs=[
        pl.BlockSpec((NC, BLK, H), lambda i: (0, i, 0)),
        pl.BlockSpec((BLK, H), lambda i: (i, 0)),
        pl.BlockSpec((BLK, 1), lambda i: (i, 0)),
        pl.BlockSpec((1, H), lambda i: (0, 0)),
        pl.BlockSpec((H, H), lambda i: (0, 0)),
    ],
    out_specs=pl.BlockSpec((BLK, H), lambda i: (i, 0)),
    out_shape=jax.ShapeDtypeStruct((N_PAD, H), jnp.float32),
)


def _final_body(acc_ref, u_ref, dis_ref, b_ref, batch_ref,
                wm0_ref, bm0_ref, wm1_ref, bm1_ref,
                out_ref, sums_ref, cnt_ref):
    i = pl.program_id(0)
    dis = dis_ref[...]
    h = jnp.maximum(dis * (acc_ref[0] + acc_ref[1] + u_ref[...]) + b_ref[...],
                    0.0)
    gids = lax.broadcasted_iota(jnp.int32, (BLK, G), 1)
    onehot = (batch_ref[...] == gids).astype(jnp.float32)
    ps = lax.dot_general(onehot, h, (((0,), (0,)), ((), ())),
                         preferred_element_type=jnp.float32,
                         precision=lax.Precision.HIGHEST)
    cs = lax.dot_general(onehot, jnp.ones((BLK, 1), jnp.float32),
                         (((0,), (0,)), ((), ())),
                         preferred_element_type=jnp.float32,
                         precision=lax.Precision.HIGHEST)

    @pl.when(i == 0)
    def _():
        sums_ref[...] = jnp.zeros_like(sums_ref)
        cnt_ref[...] = jnp.zeros_like(cnt_ref)

    sums_ref[...] += ps
    cnt_ref[...] += cs

    @pl.when(i == pl.num_programs(0) - 1)
    def _():
        pooled = sums_ref[...] / jnp.maximum(cnt_ref[...], 1.0)
        hm = jnp.maximum(
            jnp.dot(pooled, wm0_ref[...],
                    preferred_element_type=jnp.float32) + bm0_ref[...], 0.0)
        out_ref[...] = jnp.dot(hm, wm1_ref[...],
                               preferred_element_type=jnp.float32) + bm1_ref[...]


_final = pl.pallas_call(
    _final_body,
    grid=(NBLK,),
    in_specs=[
        pl.BlockSpec((NC, BLK, H), lambda i: (0, i, 0)),
        pl.BlockSpec((BLK, H), lambda i: (i, 0)),
        pl.BlockSpec((BLK, 1), lambda i: (i, 0)),
        pl.BlockSpec((1, H), lambda i: (0, 0)),
        pl.BlockSpec((BLK, 1), lambda i: (i, 0)),
        pl.BlockSpec((H, H), lambda i: (0, 0)),
        pl.BlockSpec((1, H), lambda i: (0, 0)),
        pl.BlockSpec((H, C), lambda i: (0, 0)),
        pl.BlockSpec((1, C), lambda i: (0, 0)),
    ],
    out_specs=pl.BlockSpec((G, C), lambda i: (0, 0)),
    out_shape=jax.ShapeDtypeStruct((G, C), jnp.float32),
    scratch_shapes=[
        pltpu.VMEM((G, H), jnp.float32),
        pltpu.VMEM((G, 1), jnp.float32),
    ],
)


def kernel(x, edge_index, batch, W0, b0, W1, b1, W2, b2, Wm0, bm0, Wm1, bm1):
    src = edge_index[0].astype(jnp.int32)
    dst = edge_index[1].astype(jnp.int32)
    pad_e = NW * EW - E
    pad_i = jnp.arange(pad_e, dtype=jnp.int32)
    src_p = jnp.concatenate(
        [src, pad_i % N]).reshape(NW, NB, CPB, CH)
    dst_p = jnp.concatenate(
        [dst, DUMP + pad_i % (N_PAD - N)]).reshape(NW, NB, CPB, CH)
    x_p = jnp.pad(x, ((0, N_PAD - N), (0, 0)))
    batch_p = jnp.concatenate(
        [batch.astype(jnp.int32),
         jnp.full((N_PAD - N,), G, jnp.int32)]).reshape(N_PAD, 1)
    zer = jnp.zeros((CH, H), jnp.float32)

    deg_parts = _deg_sc(dst_p.reshape(NW, EW))
    u0, dis = _prep(deg_parts, x_p, W0)
    acc = _seg_sc(u0, src_p, dst_p, zer)
    u1 = _merge(acc, u0, dis, b0.reshape(1, H), W1)
    acc = _seg_sc(u1, src_p, dst_p, zer)
    u2 = _merge(acc, u1, dis, b1.reshape(1, H), W2)
    acc = _seg_sc(u2, src_p, dst_p, zer)
    out = _final(acc, u2, dis, b2.reshape(1, H), batch_p,
                 Wm0, bm0.reshape(1, H), Wm1, bm1.reshape(1, C))
    return out
